# baseline (speedup 1.0000x reference)
import jax
import jax.numpy as jnp
from jax.experimental import pallas as pl
from jax.experimental.pallas import tpu as pltpu

Q = 16384
K = 1000
D = 16
KP = 1024
BQ = 1024
GRID = Q // BQ
NCH = KP // 8


def _body(c_ref, xt_ref, out_ref, cm2_ref, c2b_ref, x2_ref, mm_ref):
    i = pl.program_id(0)

    @pl.when(i == 0)
    def _prep():
        c = c_ref[...]
        c2 = jnp.sum(c * c, axis=1, keepdims=True)
        c2b_ref[...] = jnp.broadcast_to(c2, (KP, BQ))
        cm2_ref[...] = -2.0 * c

    j = (i - 1) % 2
    x2b = jnp.broadcast_to(x2_ref[j], (8, BQ))
    big = jnp.full((8, BQ), 3.0e38, jnp.float32)
    curmin = big
    curch = jnp.zeros((8, BQ), jnp.float32)
    for r in range(NCH):
        t = x2b + c2b_ref[pl.ds(8 * r, 8), :]
        d2 = t + mm_ref[j, pl.ds(8 * r, 8), :]
        upd = d2 < curmin
        curmin = jnp.minimum(curmin, d2)
        curch = jnp.where(upd, float(r), curch)
    m = jnp.min(curmin, axis=0, keepdims=True)
    mc = jnp.maximum(m, 0.0)
    srow = jax.lax.broadcasted_iota(jnp.int32, (8, BQ), 0).astype(jnp.float32)
    cand = jnp.where(curmin <= mc, curch * 8.0 + srow, float(KP))
    out_ref[0, 0, :] = jnp.min(cand, axis=0).astype(jnp.int32)

    xt = xt_ref[...]
    x2_ref[i % 2] = jnp.sum(xt * xt, axis=0, keepdims=True)
    mm_ref[i % 2] = jnp.dot(cm2_ref[...], xt,
                            preferred_element_type=jnp.float32)


@jax.jit
def kernel(features, cluster_centers):
    pad = jnp.full((KP - K, D), 1e17, dtype=cluster_centers.dtype)
    c = jnp.concatenate([cluster_centers, pad], axis=0)
    xt = features.T

    out = pl.pallas_call(
        _body,
        grid=(GRID + 1,),
        in_specs=[
            pl.BlockSpec((KP, D), lambda i: (0, 0)),
            pl.BlockSpec((D, BQ), lambda i: (0, jnp.minimum(i, GRID - 1))),
        ],
        out_specs=pl.BlockSpec((1, 1, BQ), lambda i: (jnp.maximum(i - 1, 0), 0, 0)),
        out_shape=jax.ShapeDtypeStruct((GRID, 1, BQ), jnp.int32),
        scratch_shapes=[
            pltpu.VMEM((KP, D), jnp.float32),
            pltpu.VMEM((KP, BQ), jnp.float32),
            pltpu.VMEM((2, 1, BQ), jnp.float32),
            pltpu.VMEM((2, KP, BQ), jnp.float32),
        ],
    )(c, xt)
    return out.reshape(Q)


# speedup vs baseline: 1.1127x; 1.1127x over previous
import jax
import jax.numpy as jnp
from jax.experimental import pallas as pl
from jax.experimental.pallas import tpu as pltpu

Q = 16384
K = 1000
D = 16
KP = 1024
BQ = 1024
GRID = Q // BQ
NCH = KP // 8


def _body(c_ref, xt_ref, out_ref, cm2_ref, c2b_ref, x2_ref, mm_ref):
    i = pl.program_id(0)

    @pl.when(i == 0)
    def _prep():
        c = c_ref[...]
        c2 = jnp.sum(c * c, axis=1, keepdims=True)
        c2b_ref[...] = jnp.broadcast_to(c2, (KP, BQ))
        cm2_ref[...] = -2.0 * c

    xt = xt_ref[...]
    x2_ref[i % 2] = jnp.sum(xt * xt, axis=0, keepdims=True)
    mm_ref[i % 2] = jnp.dot(cm2_ref[...], xt,
                            preferred_element_type=jnp.float32)

    j = (i - 1) % 2
    x2b = jnp.broadcast_to(x2_ref[j], (8, BQ))
    big = jnp.full((8, BQ), 3.0e38, jnp.float32)
    curmin = big
    curch = jnp.zeros((8, BQ), jnp.float32)
    for r in range(NCH):
        t = x2b + c2b_ref[pl.ds(8 * r, 8), :]
        d2 = t + mm_ref[j, pl.ds(8 * r, 8), :]
        upd = d2 < curmin
        curmin = jnp.minimum(curmin, d2)
        curch = jnp.where(upd, float(r), curch)
    m = jnp.min(curmin, axis=0, keepdims=True)
    mc = jnp.maximum(m, 0.0)
    srow = jax.lax.broadcasted_iota(jnp.int32, (8, BQ), 0).astype(jnp.float32)
    cand = jnp.where(curmin <= mc, curch * 8.0 + srow, float(KP))
    out_ref[0, 0, :] = jnp.min(cand, axis=0).astype(jnp.int32)


@jax.jit
def kernel(features, cluster_centers):
    pad = jnp.full((KP - K, D), 1e17, dtype=cluster_centers.dtype)
    c = jnp.concatenate([cluster_centers, pad], axis=0)
    xt = features.T

    out = pl.pallas_call(
        _body,
        grid=(GRID + 1,),
        in_specs=[
            pl.BlockSpec((KP, D), lambda i: (0, 0)),
            pl.BlockSpec((D, BQ), lambda i: (0, jnp.minimum(i, GRID - 1))),
        ],
        out_specs=pl.BlockSpec((1, 1, BQ), lambda i: (jnp.maximum(i - 1, 0), 0, 0)),
        out_shape=jax.ShapeDtypeStruct((GRID, 1, BQ), jnp.int32),
        scratch_shapes=[
            pltpu.VMEM((KP, D), jnp.float32),
            pltpu.VMEM((KP, BQ), jnp.float32),
            pltpu.VMEM((2, 1, BQ), jnp.float32),
            pltpu.VMEM((2, KP, BQ), jnp.float32),
        ],
    )(c, xt)
    return out.reshape(Q)


# speedup vs baseline: 1.3160x; 1.1826x over previous
import jax
import jax.numpy as jnp
from jax.experimental import pallas as pl
from jax.experimental.pallas import tpu as pltpu

Q = 16384
K = 1000
D = 16
KP = 1024
BQ = 1024
GRID = Q // BQ
NCH = KP // 8


def _body(c_ref, xt_ref, out_ref, cm2_ref, c2_ref, x2_ref, mm_ref):
    i = pl.program_id(0)

    @pl.when(i == 0)
    def _prep():
        c = c_ref[...]
        c2_ref[...] = jnp.sum(c * c, axis=1, keepdims=True)
        cm2_ref[...] = -2.0 * c

    xt = xt_ref[...]
    x2_ref[i % 2] = jnp.sum(xt * xt, axis=0, keepdims=True)
    mm_ref[i % 2] = jnp.dot(cm2_ref[...], xt,
                            preferred_element_type=jnp.float32)

    j = (i - 1) % 2
    x2b = jnp.broadcast_to(x2_ref[j], (8, BQ))
    big = jnp.full((8, BQ), 3.0e38, jnp.float32)
    cm = [big, big]
    cc = [jnp.zeros((8, BQ), jnp.float32)] * 2
    for r in range(NCH):
        p = r & 1
        t = x2b + c2_ref[pl.ds(8 * r, 8), :]
        d2 = t + mm_ref[j, pl.ds(8 * r, 8), :]
        upd = d2 < cm[p]
        cm[p] = jnp.minimum(cm[p], d2)
        cc[p] = jnp.where(upd, float(r), cc[p])
    curmin = jnp.minimum(cm[0], cm[1])
    m = jnp.min(curmin, axis=0, keepdims=True)
    mc = jnp.maximum(m, 0.0)
    srow = jax.lax.broadcasted_iota(jnp.int32, (8, BQ), 0).astype(jnp.float32)
    big_idx = jnp.full((8, BQ), float(KP), jnp.float32)
    cand0 = jnp.where(cm[0] <= mc, cc[0] * 8.0 + srow, big_idx)
    cand1 = jnp.where(cm[1] <= mc, cc[1] * 8.0 + srow, big_idx)
    cand = jnp.minimum(cand0, cand1)
    out_ref[0, 0, :] = jnp.min(cand, axis=0).astype(jnp.int32)


@jax.jit
def kernel(features, cluster_centers):
    pad = jnp.full((KP - K, D), 1e17, dtype=cluster_centers.dtype)
    c = jnp.concatenate([cluster_centers, pad], axis=0)
    xt = features.T

    out = pl.pallas_call(
        _body,
        grid=(GRID + 1,),
        in_specs=[
            pl.BlockSpec((KP, D), lambda i: (0, 0)),
            pl.BlockSpec((D, BQ), lambda i: (0, jnp.minimum(i, GRID - 1))),
        ],
        out_specs=pl.BlockSpec((1, 1, BQ), lambda i: (jnp.maximum(i - 1, 0), 0, 0)),
        out_shape=jax.ShapeDtypeStruct((GRID, 1, BQ), jnp.int32),
        scratch_shapes=[
            pltpu.VMEM((KP, D), jnp.float32),
            pltpu.VMEM((KP, 1), jnp.float32),
            pltpu.VMEM((2, 1, BQ), jnp.float32),
            pltpu.VMEM((2, KP, BQ), jnp.float32),
        ],
    )(c, xt)
    return out.reshape(Q)
